# Initial kernel scaffold; baseline (speedup 1.0000x reference)
import jax
import jax.numpy as jnp
from jax.experimental import pallas as pl


def kernel(x, edge_index, params):
    raise NotImplementedError("write your pallas kernel here")


# speedup vs baseline: 4.2529x; 4.2529x over previous
import functools

import jax
import jax.numpy as jnp
from jax import lax
from jax.experimental import pallas as pl
from jax.experimental.pallas import tpu as pltpu
from jax.experimental.pallas import tpu_sc as plsc

_N = 10000
_E = 320000
_IN = 128
_H = 64
_NLAYERS = 4
_EPS = 1e-5

_NC = 2
_NS = 16
_NW = _NC * _NS
_BE = 128
_NB = 80
_EPW = _NB * _BE
_EPAD = _NW * _EPW
_NACC = 10112
_RPT = _NACC // _NS


def _sc_aggr_body(m_hbm, src_hbm, dst_hbm, zro_hbm, out_hbm,
                  src_v, dst_v, rows_v, acc_sh, gsem):
    c = lax.axis_index("c")
    s = lax.axis_index("s")
    wid = c * _NS + s
    pltpu.sync_copy(src_hbm.at[wid], src_v)
    pltpu.sync_copy(dst_hbm.at[wid], dst_v)
    pltpu.sync_copy(zro_hbm.at[pl.ds(s * _RPT, _RPT)],
                    acc_sh.at[pl.ds(s * _RPT, _RPT)])
    plsc.subcore_barrier()

    def body(j, carry):
        pltpu.async_copy(m_hbm.at[src_v.at[j]], rows_v.at[0], gsem).wait()
        pltpu.sync_copy(rows_v.at[0], acc_sh.at[dst_v.at[j]], add=True)
        return carry

    lax.fori_loop(0, _NB, body, 0)
    plsc.subcore_barrier()
    pltpu.sync_copy(acc_sh.at[pl.ds(s * _RPT, _RPT)],
                    out_hbm.at[c, pl.ds(s * _RPT, _RPT)])


@functools.cache
def _sc_aggr():
    return pl.kernel(
        _sc_aggr_body,
        out_type=jax.ShapeDtypeStruct((_NC, _NACC, _H), jnp.float32),
        mesh=plsc.VectorSubcoreMesh(core_axis_name="c", subcore_axis_name="s"),
        scratch_types=[
            pltpu.VMEM((_NB, _BE), jnp.int32),
            pltpu.VMEM((_NB, _BE), jnp.int32),
            pltpu.VMEM((1, _BE, _H), jnp.float32),
            pltpu.VMEM_SHARED((_NACC, _H), jnp.float32),
            pltpu.SemaphoreType.DMA,
        ],
        compiler_params=pltpu.CompilerParams(use_tc_tiling_on_sc=False),
    )


def _dot(a, b):
    return jnp.dot(a, b, preferred_element_type=jnp.float32)


def _tc_in_body(x_ref, wi_ref, bi_ref, w1_ref, b1_ref, w2_ref, b2_ref,
                h_ref, m_ref):
    h = jnp.maximum(_dot(x_ref[...], wi_ref[...]) + bi_ref[...], 0.0)
    h_ref[...] = h
    t = jnp.maximum(_dot(h, w1_ref[...]) + b1_ref[...], 0.0)
    m_ref[...] = _dot(t, w2_ref[...]) + b2_ref[...]


def _gru_bn(h, p0, p1, wih_r, wih_z, wih_n, bih_r, bih_z, bih_n,
            whh_r, whh_z, whh_n, bhh_r, bhh_z, bhh_n, gam, bet):
    aggr = p0[:_N] + p1[:_N]
    r = jax.nn.sigmoid(_dot(aggr, wih_r) + bih_r + _dot(h, whh_r) + bhh_r)
    z = jax.nn.sigmoid(_dot(aggr, wih_z) + bih_z + _dot(h, whh_z) + bhh_z)
    n = jnp.tanh(_dot(aggr, wih_n) + bih_n + r * (_dot(h, whh_n) + bhh_n))
    hn = (1.0 - z) * n + z * h
    mean = jnp.mean(hn, axis=0, keepdims=True)
    var = jnp.mean((hn - mean) ** 2, axis=0, keepdims=True)
    return (hn - mean) / jnp.sqrt(var + _EPS) * gam + bet + h


def _tc_up_body(h_ref, p0_ref, p1_ref, wih_r_ref, wih_z_ref, wih_n_ref,
                bih_r_ref, bih_z_ref, bih_n_ref, whh_r_ref, whh_z_ref,
                whh_n_ref, bhh_r_ref, bhh_z_ref, bhh_n_ref, gam_ref, bet_ref,
                w1_ref, b1_ref, w2_ref, b2_ref, h_out_ref, m_out_ref):
    hb = _gru_bn(h_ref[...], p0_ref[...], p1_ref[...],
                 wih_r_ref[...], wih_z_ref[...], wih_n_ref[...],
                 bih_r_ref[...], bih_z_ref[...], bih_n_ref[...],
                 whh_r_ref[...], whh_z_ref[...], whh_n_ref[...],
                 bhh_r_ref[...], bhh_z_ref[...], bhh_n_ref[...],
                 gam_ref[...], bet_ref[...])
    h_out_ref[...] = hb
    t = jnp.maximum(_dot(hb, w1_ref[...]) + b1_ref[...], 0.0)
    m_out_ref[...] = _dot(t, w2_ref[...]) + b2_ref[...]


def _tc_last_body(h_ref, p0_ref, p1_ref, wih_r_ref, wih_z_ref, wih_n_ref,
                  bih_r_ref, bih_z_ref, bih_n_ref, whh_r_ref, whh_z_ref,
                  whh_n_ref, bhh_r_ref, bhh_z_ref, bhh_n_ref, gam_ref,
                  bet_ref, wo_ref, bo_ref, out_ref):
    hb = _gru_bn(h_ref[...], p0_ref[...], p1_ref[...],
                 wih_r_ref[...], wih_z_ref[...], wih_n_ref[...],
                 bih_r_ref[...], bih_z_ref[...], bih_n_ref[...],
                 whh_r_ref[...], whh_z_ref[...], whh_n_ref[...],
                 bhh_r_ref[...], bhh_z_ref[...], bhh_n_ref[...],
                 gam_ref[...], bet_ref[...])
    out_ref[...] = _dot(hb, wo_ref[...]) + bo_ref[...]


def _f32_out(shape):
    return jax.ShapeDtypeStruct(shape, jnp.float32)


@functools.cache
def _tc_in():
    return pl.pallas_call(
        _tc_in_body,
        out_shape=[_f32_out((_N, _H)), _f32_out((_N, _H))],
    )


@functools.cache
def _tc_up():
    return pl.pallas_call(
        _tc_up_body,
        out_shape=[_f32_out((_N, _H)), _f32_out((_N, _H))],
    )


@functools.cache
def _tc_last():
    return pl.pallas_call(
        _tc_last_body,
        out_shape=_f32_out((_N, 1)),
    )


def _split_gru(lp):
    w_ih, w_hh = lp["gru_w_ih"], lp["gru_w_hh"]
    b_ih, b_hh = lp["gru_b_ih"], lp["gru_b_hh"]
    parts = []
    for k in range(3):
        parts.append(w_ih[k * _H:(k + 1) * _H].T)
    for k in range(3):
        parts.append(b_ih[k * _H:(k + 1) * _H].reshape(1, _H))
    for k in range(3):
        parts.append(w_hh[k * _H:(k + 1) * _H].T)
    for k in range(3):
        parts.append(b_hh[k * _H:(k + 1) * _H].reshape(1, _H))
    return (parts[0], parts[1], parts[2], parts[3], parts[4], parts[5],
            parts[6], parts[7], parts[8], parts[9], parts[10], parts[11],
            lp["bn_gamma"].reshape(1, _H), lp["bn_beta"].reshape(1, _H))


def kernel(x, edge_index, params):
    src = edge_index[0]
    dst = edge_index[1]
    pad = _EPAD - _E
    src_p = jnp.concatenate(
        [src, jnp.zeros((pad,), jnp.int32)]).reshape(_NW, _NB, _BE)
    dst_p = jnp.concatenate(
        [dst, jnp.full((pad,), _N, jnp.int32)]).reshape(_NW, _NB, _BE)
    zeros_acc = jnp.zeros((_NACC, _H), jnp.float32)

    layers = params["layers"]
    lp0 = layers[0]
    h, m = _tc_in()(
        x, params["input_w"], params["input_b"].reshape(1, _H),
        lp0["msg_w1"], lp0["msg_b1"].reshape(1, _H),
        lp0["msg_w2"], lp0["msg_b2"].reshape(1, _H))

    out = None
    for li in range(_NLAYERS):
        lp = layers[li]
        part = _sc_aggr()(m, src_p, dst_p, zeros_acc)
        p0, p1 = part[0], part[1]
        gru = _split_gru(lp)
        if li < _NLAYERS - 1:
            lpn = layers[li + 1]
            h, m = _tc_up()(
                h, p0, p1, *gru,
                lpn["msg_w1"], lpn["msg_b1"].reshape(1, _H),
                lpn["msg_w2"], lpn["msg_b2"].reshape(1, _H))
        else:
            out = _tc_last()(
                h, p0, p1, *gru,
                params["out_w"], params["out_b"].reshape(1, 1))
    return jnp.squeeze(out, axis=-1)


# speedup vs baseline: 5.0084x; 1.1776x over previous
import functools

import jax
import jax.numpy as jnp
from jax import lax
from jax.experimental import pallas as pl
from jax.experimental.pallas import tpu as pltpu
from jax.experimental.pallas import tpu_sc as plsc

_N = 10000
_E = 320000
_IN = 128
_H = 64
_NLAYERS = 4
_EPS = 1e-5

_NC = 2
_NS = 16
_NW = _NC * _NS
_BE = 128
_NB = 80
_EPW = _NB * _BE
_EPAD = _NW * _EPW
_NACC = 10112
_RPT = _NACC // _NS
_K = 8


def _sc_aggr_body(m_hbm, src_hbm, dst_hbm, zro_hbm, out_hbm,
                  src_v, dst_v, rows_v, acc_sh, gsem):
    c = lax.axis_index("c")
    s = lax.axis_index("s")
    wid = c * _NS + s
    pltpu.sync_copy(src_hbm.at[wid], src_v)
    pltpu.sync_copy(dst_hbm.at[wid], dst_v)
    pltpu.sync_copy(zro_hbm.at[pl.ds(s * _RPT, _RPT)],
                    acc_sh.at[pl.ds(s * _RPT, _RPT)])
    plsc.subcore_barrier()

    for b in range(_K):
        pltpu.async_copy(m_hbm.at[src_v.at[b]], rows_v.at[b], gsem.at[b])

    def chunk(i, carry):
        for b in range(_K):
            j = i * _K + b
            pltpu.make_async_copy(m_hbm.at[src_v.at[j]], rows_v.at[b],
                                  gsem.at[b]).wait()
            pltpu.sync_copy(rows_v.at[b], acc_sh.at[dst_v.at[j]], add=True)

            @pl.when(j + _K < _NB)
            def _refill():
                pltpu.async_copy(m_hbm.at[src_v.at[j + _K]], rows_v.at[b],
                                 gsem.at[b])
        return carry

    lax.fori_loop(0, _NB // _K, chunk, 0)
    plsc.subcore_barrier()
    pltpu.sync_copy(acc_sh.at[pl.ds(s * _RPT, _RPT)],
                    out_hbm.at[c, pl.ds(s * _RPT, _RPT)])


@functools.cache
def _sc_aggr():
    return pl.kernel(
        _sc_aggr_body,
        out_type=jax.ShapeDtypeStruct((_NC, _NACC, _H), jnp.float32),
        mesh=plsc.VectorSubcoreMesh(core_axis_name="c", subcore_axis_name="s"),
        scratch_types=[
            pltpu.VMEM((_NB, _BE), jnp.int32),
            pltpu.VMEM((_NB, _BE), jnp.int32),
            pltpu.VMEM((_K, _BE, _H), jnp.float32),
            pltpu.VMEM_SHARED((_NACC, _H), jnp.float32),
            pltpu.SemaphoreType.DMA((_K,)),
        ],
        compiler_params=pltpu.CompilerParams(use_tc_tiling_on_sc=False),
    )


def _dot(a, b):
    return jnp.dot(a, b, preferred_element_type=jnp.float32)


def _tc_in_body(x_ref, wi_ref, bi_ref, w1_ref, b1_ref, w2_ref, b2_ref,
                h_ref, m_ref):
    h = jnp.maximum(_dot(x_ref[...], wi_ref[...]) + bi_ref[...], 0.0)
    h_ref[...] = h
    t = jnp.maximum(_dot(h, w1_ref[...]) + b1_ref[...], 0.0)
    m_ref[...] = _dot(t, w2_ref[...]) + b2_ref[...]


def _gru_bn(h, p0, p1, wih_r, wih_z, wih_n, bih_r, bih_z, bih_n,
            whh_r, whh_z, whh_n, bhh_r, bhh_z, bhh_n, gam, bet):
    aggr = p0[:_N] + p1[:_N]
    r = jax.nn.sigmoid(_dot(aggr, wih_r) + bih_r + _dot(h, whh_r) + bhh_r)
    z = jax.nn.sigmoid(_dot(aggr, wih_z) + bih_z + _dot(h, whh_z) + bhh_z)
    n = jnp.tanh(_dot(aggr, wih_n) + bih_n + r * (_dot(h, whh_n) + bhh_n))
    hn = (1.0 - z) * n + z * h
    mean = jnp.mean(hn, axis=0, keepdims=True)
    var = jnp.mean((hn - mean) ** 2, axis=0, keepdims=True)
    return (hn - mean) / jnp.sqrt(var + _EPS) * gam + bet + h


def _tc_up_body(h_ref, p0_ref, p1_ref, wih_r_ref, wih_z_ref, wih_n_ref,
                bih_r_ref, bih_z_ref, bih_n_ref, whh_r_ref, whh_z_ref,
                whh_n_ref, bhh_r_ref, bhh_z_ref, bhh_n_ref, gam_ref, bet_ref,
                w1_ref, b1_ref, w2_ref, b2_ref, h_out_ref, m_out_ref):
    hb = _gru_bn(h_ref[...], p0_ref[...], p1_ref[...],
                 wih_r_ref[...], wih_z_ref[...], wih_n_ref[...],
                 bih_r_ref[...], bih_z_ref[...], bih_n_ref[...],
                 whh_r_ref[...], whh_z_ref[...], whh_n_ref[...],
                 bhh_r_ref[...], bhh_z_ref[...], bhh_n_ref[...],
                 gam_ref[...], bet_ref[...])
    h_out_ref[...] = hb
    t = jnp.maximum(_dot(hb, w1_ref[...]) + b1_ref[...], 0.0)
    m_out_ref[...] = _dot(t, w2_ref[...]) + b2_ref[...]


def _tc_last_body(h_ref, p0_ref, p1_ref, wih_r_ref, wih_z_ref, wih_n_ref,
                  bih_r_ref, bih_z_ref, bih_n_ref, whh_r_ref, whh_z_ref,
                  whh_n_ref, bhh_r_ref, bhh_z_ref, bhh_n_ref, gam_ref,
                  bet_ref, wo_ref, bo_ref, out_ref):
    hb = _gru_bn(h_ref[...], p0_ref[...], p1_ref[...],
                 wih_r_ref[...], wih_z_ref[...], wih_n_ref[...],
                 bih_r_ref[...], bih_z_ref[...], bih_n_ref[...],
                 whh_r_ref[...], whh_z_ref[...], whh_n_ref[...],
                 bhh_r_ref[...], bhh_z_ref[...], bhh_n_ref[...],
                 gam_ref[...], bet_ref[...])
    out_ref[...] = _dot(hb, wo_ref[...]) + bo_ref[...]


def _f32_out(shape):
    return jax.ShapeDtypeStruct(shape, jnp.float32)


@functools.cache
def _tc_in():
    return pl.pallas_call(
        _tc_in_body,
        out_shape=[_f32_out((_N, _H)), _f32_out((_N, _H))],
    )


@functools.cache
def _tc_up():
    return pl.pallas_call(
        _tc_up_body,
        out_shape=[_f32_out((_N, _H)), _f32_out((_N, _H))],
    )


@functools.cache
def _tc_last():
    return pl.pallas_call(
        _tc_last_body,
        out_shape=_f32_out((_N, 1)),
    )


def _split_gru(lp):
    w_ih, w_hh = lp["gru_w_ih"], lp["gru_w_hh"]
    b_ih, b_hh = lp["gru_b_ih"], lp["gru_b_hh"]
    parts = []
    for k in range(3):
        parts.append(w_ih[k * _H:(k + 1) * _H].T)
    for k in range(3):
        parts.append(b_ih[k * _H:(k + 1) * _H].reshape(1, _H))
    for k in range(3):
        parts.append(w_hh[k * _H:(k + 1) * _H].T)
    for k in range(3):
        parts.append(b_hh[k * _H:(k + 1) * _H].reshape(1, _H))
    return (parts[0], parts[1], parts[2], parts[3], parts[4], parts[5],
            parts[6], parts[7], parts[8], parts[9], parts[10], parts[11],
            lp["bn_gamma"].reshape(1, _H), lp["bn_beta"].reshape(1, _H))


def kernel(x, edge_index, params):
    src = edge_index[0]
    dst = edge_index[1]
    pad = _EPAD - _E
    src_p = jnp.concatenate(
        [src, jnp.zeros((pad,), jnp.int32)]).reshape(_NW, _NB, _BE)
    dst_p = jnp.concatenate(
        [dst, jnp.full((pad,), _N, jnp.int32)]).reshape(_NW, _NB, _BE)
    zeros_acc = jnp.zeros((_NACC, _H), jnp.float32)

    layers = params["layers"]
    lp0 = layers[0]
    h, m = _tc_in()(
        x, params["input_w"], params["input_b"].reshape(1, _H),
        lp0["msg_w1"], lp0["msg_b1"].reshape(1, _H),
        lp0["msg_w2"], lp0["msg_b2"].reshape(1, _H))

    out = None
    for li in range(_NLAYERS):
        lp = layers[li]
        part = _sc_aggr()(m, src_p, dst_p, zeros_acc)
        p0, p1 = part[0], part[1]
        gru = _split_gru(lp)
        if li < _NLAYERS - 1:
            lpn = layers[li + 1]
            h, m = _tc_up()(
                h, p0, p1, *gru,
                lpn["msg_w1"], lpn["msg_b1"].reshape(1, _H),
                lpn["msg_w2"], lpn["msg_b2"].reshape(1, _H))
        else:
            out = _tc_last()(
                h, p0, p1, *gru,
                params["out_w"], params["out_b"].reshape(1, 1))
    return jnp.squeeze(out, axis=-1)


# speedup vs baseline: 14.2666x; 2.8485x over previous
import functools

import jax
import jax.numpy as jnp
from jax import lax
from jax.experimental import pallas as pl
from jax.experimental.pallas import tpu as pltpu
from jax.experimental.pallas import tpu_sc as plsc

_N = 10000
_E = 320000
_IN = 128
_H = 64
_NLAYERS = 4
_EPS = 1e-5

_NC = 2
_NS = 16
_NW = _NC * _NS
_BE = 128
_NB = 80
_EPW = _NB * _BE
_EPAD = _NW * _EPW
_NACC = 10112
_RPT = _NACC // _NS
_K = 8


def _sc_aggr_body(m_hbm, src_hbm, dst_hbm, zro_hbm, out_hbm,
                  src_v, dst_v, rows_v, acc_sh, gsem):
    c = lax.axis_index("c")
    s = lax.axis_index("s")
    wid = c * _NS + s
    pltpu.sync_copy(src_hbm.at[wid], src_v)
    pltpu.sync_copy(dst_hbm.at[wid], dst_v)
    pltpu.sync_copy(zro_hbm.at[pl.ds(s * _RPT, _RPT)],
                    acc_sh.at[pl.ds(s * _RPT, _RPT)])
    plsc.subcore_barrier()

    for b in range(_K):
        pltpu.async_copy(m_hbm.at[src_v.at[b]], rows_v.at[b], gsem.at[b])

    def chunk(i, carry):
        for b in range(_K):
            j = i * _K + b
            pltpu.make_async_copy(m_hbm.at[src_v.at[j]], rows_v.at[b],
                                  gsem.at[b]).wait()
            pltpu.sync_copy(rows_v.at[b], acc_sh.at[dst_v.at[j]], add=True)

            @pl.when(j + _K < _NB)
            def _refill():
                pltpu.async_copy(m_hbm.at[src_v.at[j + _K]], rows_v.at[b],
                                 gsem.at[b])
        return carry

    lax.fori_loop(0, _NB // _K, chunk, 0)
    plsc.subcore_barrier()
    pltpu.sync_copy(acc_sh.at[pl.ds(s * _RPT, _RPT)],
                    out_hbm.at[c, pl.ds(s * _RPT, _RPT)])


@functools.cache
def _sc_aggr():
    return pl.kernel(
        _sc_aggr_body,
        out_type=jax.ShapeDtypeStruct((_NC, _NACC, _H), jnp.float32),
        mesh=plsc.VectorSubcoreMesh(core_axis_name="c", subcore_axis_name="s"),
        scratch_types=[
            pltpu.VMEM((_NB, _BE), jnp.int32),
            pltpu.VMEM((_NB, _BE), jnp.int32),
            pltpu.VMEM((_K, _BE, _H), jnp.float32),
            pltpu.VMEM_SHARED((_NACC, _H), jnp.float32),
            pltpu.SemaphoreType.DMA((_K,)),
        ],
        compiler_params=pltpu.CompilerParams(use_tc_tiling_on_sc=False),
    )


def _dot(a, b):
    return jnp.dot(a, b, preferred_element_type=jnp.float32)


def _tc_in_body(x_ref, wi_ref, bi_ref, w1_ref, b1_ref, w2_ref, b2_ref,
                h_ref, m_ref):
    h = jnp.maximum(_dot(x_ref[...], wi_ref[...]) + bi_ref[...], 0.0)
    h_ref[...] = h
    t = jnp.maximum(_dot(h, w1_ref[...]) + b1_ref[...], 0.0)
    m_ref[...] = _dot(t, w2_ref[...]) + b2_ref[...]


def _gru_bn(h, p0, p1, wih_r, wih_z, wih_n, bih_r, bih_z, bih_n,
            whh_r, whh_z, whh_n, bhh_r, bhh_z, bhh_n, gam, bet):
    aggr = p0[:_N] + p1[:_N]
    r = jax.nn.sigmoid(_dot(aggr, wih_r) + bih_r + _dot(h, whh_r) + bhh_r)
    z = jax.nn.sigmoid(_dot(aggr, wih_z) + bih_z + _dot(h, whh_z) + bhh_z)
    n = jnp.tanh(_dot(aggr, wih_n) + bih_n + r * (_dot(h, whh_n) + bhh_n))
    hn = (1.0 - z) * n + z * h
    mean = jnp.mean(hn, axis=0, keepdims=True)
    var = jnp.mean((hn - mean) ** 2, axis=0, keepdims=True)
    return (hn - mean) / jnp.sqrt(var + _EPS) * gam + bet + h


def _tc_up_body(h_ref, p0_ref, p1_ref, wih_r_ref, wih_z_ref, wih_n_ref,
                bih_r_ref, bih_z_ref, bih_n_ref, whh_r_ref, whh_z_ref,
                whh_n_ref, bhh_r_ref, bhh_z_ref, bhh_n_ref, gam_ref, bet_ref,
                w1_ref, b1_ref, w2_ref, b2_ref, h_out_ref, m_out_ref):
    hb = _gru_bn(h_ref[...], p0_ref[...], p1_ref[...],
                 wih_r_ref[...], wih_z_ref[...], wih_n_ref[...],
                 bih_r_ref[...], bih_z_ref[...], bih_n_ref[...],
                 whh_r_ref[...], whh_z_ref[...], whh_n_ref[...],
                 bhh_r_ref[...], bhh_z_ref[...], bhh_n_ref[...],
                 gam_ref[...], bet_ref[...])
    h_out_ref[...] = hb
    t = jnp.maximum(_dot(hb, w1_ref[...]) + b1_ref[...], 0.0)
    m_out_ref[...] = _dot(t, w2_ref[...]) + b2_ref[...]


def _tc_last_body(h_ref, p0_ref, p1_ref, wih_r_ref, wih_z_ref, wih_n_ref,
                  bih_r_ref, bih_z_ref, bih_n_ref, whh_r_ref, whh_z_ref,
                  whh_n_ref, bhh_r_ref, bhh_z_ref, bhh_n_ref, gam_ref,
                  bet_ref, wo_ref, bo_ref, out_ref):
    hb = _gru_bn(h_ref[...], p0_ref[...], p1_ref[...],
                 wih_r_ref[...], wih_z_ref[...], wih_n_ref[...],
                 bih_r_ref[...], bih_z_ref[...], bih_n_ref[...],
                 whh_r_ref[...], whh_z_ref[...], whh_n_ref[...],
                 bhh_r_ref[...], bhh_z_ref[...], bhh_n_ref[...],
                 gam_ref[...], bet_ref[...])
    out_ref[...] = _dot(hb, wo_ref[...]) + bo_ref[...]


def _f32_out(shape):
    return jax.ShapeDtypeStruct(shape, jnp.float32)


@functools.cache
def _tc_in():
    return pl.pallas_call(
        _tc_in_body,
        out_shape=[_f32_out((_N, _H)), _f32_out((_N, _H))],
    )


@functools.cache
def _tc_up():
    return pl.pallas_call(
        _tc_up_body,
        out_shape=[_f32_out((_N, _H)), _f32_out((_N, _H))],
    )


@functools.cache
def _tc_last():
    return pl.pallas_call(
        _tc_last_body,
        out_shape=_f32_out((_N, 1)),
    )


def _split_gru(lp):
    w_ih, w_hh = lp["gru_w_ih"], lp["gru_w_hh"]
    b_ih, b_hh = lp["gru_b_ih"], lp["gru_b_hh"]
    parts = []
    for k in range(3):
        parts.append(w_ih[k * _H:(k + 1) * _H].T)
    for k in range(3):
        parts.append(b_ih[k * _H:(k + 1) * _H].reshape(1, _H))
    for k in range(3):
        parts.append(w_hh[k * _H:(k + 1) * _H].T)
    for k in range(3):
        parts.append(b_hh[k * _H:(k + 1) * _H].reshape(1, _H))
    return (parts[0], parts[1], parts[2], parts[3], parts[4], parts[5],
            parts[6], parts[7], parts[8], parts[9], parts[10], parts[11],
            lp["bn_gamma"].reshape(1, _H), lp["bn_beta"].reshape(1, _H))


def kernel(x, edge_index, params):
    src = edge_index[0]
    dst = edge_index[1]
    pad = _EPAD - _E
    pad_src = jnp.arange(pad, dtype=jnp.int32) % _N
    pad_dst = _N + (jnp.arange(pad, dtype=jnp.int32) % (_NACC - _N))
    src_p = jnp.concatenate([src, pad_src]).reshape(_NW, _NB, _BE)
    dst_p = jnp.concatenate([dst, pad_dst]).reshape(_NW, _NB, _BE)
    zeros_acc = jnp.zeros((_NACC, _H), jnp.float32)

    layers = params["layers"]
    lp0 = layers[0]
    h, m = _tc_in()(
        x, params["input_w"], params["input_b"].reshape(1, _H),
        lp0["msg_w1"], lp0["msg_b1"].reshape(1, _H),
        lp0["msg_w2"], lp0["msg_b2"].reshape(1, _H))

    out = None
    for li in range(_NLAYERS):
        lp = layers[li]
        part = _sc_aggr()(m, src_p, dst_p, zeros_acc)
        p0, p1 = part[0], part[1]
        gru = _split_gru(lp)
        if li < _NLAYERS - 1:
            lpn = layers[li + 1]
            h, m = _tc_up()(
                h, p0, p1, *gru,
                lpn["msg_w1"], lpn["msg_b1"].reshape(1, _H),
                lpn["msg_w2"], lpn["msg_b2"].reshape(1, _H))
        else:
            out = _tc_last()(
                h, p0, p1, *gru,
                params["out_w"], params["out_b"].reshape(1, 1))
    return jnp.squeeze(out, axis=-1)


# speedup vs baseline: 15.2883x; 1.0716x over previous
import functools

import jax
import jax.numpy as jnp
from jax import lax
from jax.experimental import pallas as pl
from jax.experimental.pallas import tpu as pltpu
from jax.experimental.pallas import tpu_sc as plsc

_N = 10000
_E = 320000
_IN = 128
_H = 64
_NLAYERS = 4
_EPS = 1e-5

_NC = 2
_NS = 16
_NW = _NC * _NS
_BE = 128
_NB = 80
_EPW = _NB * _BE
_EPAD = _NW * _EPW
_NACC = 10112
_RPT = _NACC // _NS
_K = 8


def _sc_aggr_body(m_hbm, src_hbm, dst_hbm, zro_hbm, out_hbm,
                  src_v, dst_v, rows_v, acc_sh, gsem, ssem):
    c = lax.axis_index("c")
    s = lax.axis_index("s")
    wid = c * _NS + s
    pltpu.sync_copy(src_hbm.at[wid], src_v)
    pltpu.sync_copy(dst_hbm.at[wid], dst_v)
    pltpu.sync_copy(zro_hbm.at[pl.ds(s * _RPT, _RPT)],
                    acc_sh.at[pl.ds(s * _RPT, _RPT)])
    plsc.subcore_barrier()

    for b in range(_K):
        pltpu.async_copy(m_hbm.at[src_v.at[b]], rows_v.at[b], gsem.at[b])

    def chunk(i, carry):
        for b in range(_K):
            j = i * _K + b
            jp = j - 1
            bp = (b - 1) % _K
            pltpu.make_async_copy(m_hbm.at[src_v.at[j]], rows_v.at[b],
                                  gsem.at[b]).wait()
            pltpu.async_copy(rows_v.at[b], acc_sh.at[dst_v.at[j]],
                             ssem.at[b], add=True)

            @pl.when(jnp.logical_and(jp >= 0, jp + _K < _NB))
            def _refill():
                pltpu.make_async_copy(rows_v.at[bp], acc_sh.at[dst_v.at[jp]],
                                      ssem.at[bp]).wait()
                pltpu.async_copy(m_hbm.at[src_v.at[jp + _K]], rows_v.at[bp],
                                 gsem.at[bp])
        return carry

    lax.fori_loop(0, _NB // _K, chunk, 0)
    for b in range(_K):
        j = _NB - _K + b
        pltpu.make_async_copy(rows_v.at[b], acc_sh.at[dst_v.at[j]],
                              ssem.at[b]).wait()
    plsc.subcore_barrier()
    pltpu.sync_copy(acc_sh.at[pl.ds(s * _RPT, _RPT)],
                    out_hbm.at[c, pl.ds(s * _RPT, _RPT)])


@functools.cache
def _sc_aggr():
    return pl.kernel(
        _sc_aggr_body,
        out_type=jax.ShapeDtypeStruct((_NC, _NACC, _H), jnp.float32),
        mesh=plsc.VectorSubcoreMesh(core_axis_name="c", subcore_axis_name="s"),
        scratch_types=[
            pltpu.VMEM((_NB, _BE), jnp.int32),
            pltpu.VMEM((_NB, _BE), jnp.int32),
            pltpu.VMEM((_K, _BE, _H), jnp.float32),
            pltpu.VMEM_SHARED((_NACC, _H), jnp.float32),
            pltpu.SemaphoreType.DMA((_K,)),
            pltpu.SemaphoreType.DMA((_K,)),
        ],
        compiler_params=pltpu.CompilerParams(use_tc_tiling_on_sc=False),
    )


def _dot(a, b):
    return jnp.dot(a, b, preferred_element_type=jnp.float32)


def _tc_in_body(x_ref, wi_ref, bi_ref, w1_ref, b1_ref, w2_ref, b2_ref,
                h_ref, m_ref):
    h = jnp.maximum(_dot(x_ref[...], wi_ref[...]) + bi_ref[...], 0.0)
    h_ref[...] = h
    t = jnp.maximum(_dot(h, w1_ref[...]) + b1_ref[...], 0.0)
    m_ref[...] = _dot(t, w2_ref[...]) + b2_ref[...]


def _gru_bn(h, part, wih_r, wih_z, wih_n, bih_r, bih_z, bih_n,
            whh_r, whh_z, whh_n, bhh_r, bhh_z, bhh_n, gam, bet):
    aggr = part[0, :_N] + part[1, :_N]
    r = jax.nn.sigmoid(_dot(aggr, wih_r) + bih_r + _dot(h, whh_r) + bhh_r)
    z = jax.nn.sigmoid(_dot(aggr, wih_z) + bih_z + _dot(h, whh_z) + bhh_z)
    n = jnp.tanh(_dot(aggr, wih_n) + bih_n + r * (_dot(h, whh_n) + bhh_n))
    hn = (1.0 - z) * n + z * h
    mean = jnp.mean(hn, axis=0, keepdims=True)
    var = jnp.mean((hn - mean) ** 2, axis=0, keepdims=True)
    return (hn - mean) / jnp.sqrt(var + _EPS) * gam + bet + h


def _tc_up_body(h_ref, part_ref, wih_r_ref, wih_z_ref, wih_n_ref,
                bih_r_ref, bih_z_ref, bih_n_ref, whh_r_ref, whh_z_ref,
                whh_n_ref, bhh_r_ref, bhh_z_ref, bhh_n_ref, gam_ref, bet_ref,
                w1_ref, b1_ref, w2_ref, b2_ref, h_out_ref, m_out_ref):
    hb = _gru_bn(h_ref[...], part_ref[...],
                 wih_r_ref[...], wih_z_ref[...], wih_n_ref[...],
                 bih_r_ref[...], bih_z_ref[...], bih_n_ref[...],
                 whh_r_ref[...], whh_z_ref[...], whh_n_ref[...],
                 bhh_r_ref[...], bhh_z_ref[...], bhh_n_ref[...],
                 gam_ref[...], bet_ref[...])
    h_out_ref[...] = hb
    t = jnp.maximum(_dot(hb, w1_ref[...]) + b1_ref[...], 0.0)
    m_out_ref[...] = _dot(t, w2_ref[...]) + b2_ref[...]


def _tc_last_body(h_ref, part_ref, wih_r_ref, wih_z_ref, wih_n_ref,
                  bih_r_ref, bih_z_ref, bih_n_ref, whh_r_ref, whh_z_ref,
                  whh_n_ref, bhh_r_ref, bhh_z_ref, bhh_n_ref, gam_ref,
                  bet_ref, wo_ref, bo_ref, out_ref):
    hb = _gru_bn(h_ref[...], part_ref[...],
                 wih_r_ref[...], wih_z_ref[...], wih_n_ref[...],
                 bih_r_ref[...], bih_z_ref[...], bih_n_ref[...],
                 whh_r_ref[...], whh_z_ref[...], whh_n_ref[...],
                 bhh_r_ref[...], bhh_z_ref[...], bhh_n_ref[...],
                 gam_ref[...], bet_ref[...])
    out_ref[...] = _dot(hb, wo_ref[...]) + bo_ref[...]


def _f32_out(shape):
    return jax.ShapeDtypeStruct(shape, jnp.float32)


@functools.cache
def _tc_in():
    return pl.pallas_call(
        _tc_in_body,
        out_shape=[_f32_out((_N, _H)), _f32_out((_N, _H))],
    )


@functools.cache
def _tc_up():
    return pl.pallas_call(
        _tc_up_body,
        out_shape=[_f32_out((_N, _H)), _f32_out((_N, _H))],
    )


@functools.cache
def _tc_last():
    return pl.pallas_call(
        _tc_last_body,
        out_shape=_f32_out((_N, 1)),
    )


def _split_gru(lp):
    w_ih, w_hh = lp["gru_w_ih"], lp["gru_w_hh"]
    b_ih, b_hh = lp["gru_b_ih"], lp["gru_b_hh"]
    parts = []
    for k in range(3):
        parts.append(w_ih[k * _H:(k + 1) * _H].T)
    for k in range(3):
        parts.append(b_ih[k * _H:(k + 1) * _H].reshape(1, _H))
    for k in range(3):
        parts.append(w_hh[k * _H:(k + 1) * _H].T)
    for k in range(3):
        parts.append(b_hh[k * _H:(k + 1) * _H].reshape(1, _H))
    return (parts[0], parts[1], parts[2], parts[3], parts[4], parts[5],
            parts[6], parts[7], parts[8], parts[9], parts[10], parts[11],
            lp["bn_gamma"].reshape(1, _H), lp["bn_beta"].reshape(1, _H))


def kernel(x, edge_index, params):
    src = edge_index[0]
    dst = edge_index[1]
    pad = _EPAD - _E
    pad_src = jnp.arange(pad, dtype=jnp.int32) % _N
    pad_dst = _N + (jnp.arange(pad, dtype=jnp.int32) % (_NACC - _N))
    src_p = jnp.concatenate([src, pad_src]).reshape(_NW, _NB, _BE)
    dst_p = jnp.concatenate([dst, pad_dst]).reshape(_NW, _NB, _BE)
    zeros_acc = jnp.zeros((_NACC, _H), jnp.float32)

    layers = params["layers"]
    lp0 = layers[0]
    h, m = _tc_in()(
        x, params["input_w"], params["input_b"].reshape(1, _H),
        lp0["msg_w1"], lp0["msg_b1"].reshape(1, _H),
        lp0["msg_w2"], lp0["msg_b2"].reshape(1, _H))

    out = None
    for li in range(_NLAYERS):
        lp = layers[li]
        part = _sc_aggr()(m, src_p, dst_p, zeros_acc)
        gru = _split_gru(lp)
        if li < _NLAYERS - 1:
            lpn = layers[li + 1]
            h, m = _tc_up()(
                h, part, *gru,
                lpn["msg_w1"], lpn["msg_b1"].reshape(1, _H),
                lpn["msg_w2"], lpn["msg_b2"].reshape(1, _H))
        else:
            out = _tc_last()(
                h, part, *gru,
                params["out_w"], params["out_b"].reshape(1, 1))
    return jnp.squeeze(out, axis=-1)


# speedup vs baseline: 15.5618x; 1.0179x over previous
import functools

import jax
import jax.numpy as jnp
from jax import lax
from jax.experimental import pallas as pl
from jax.experimental.pallas import tpu as pltpu
from jax.experimental.pallas import tpu_sc as plsc

_N = 10000
_E = 320000
_IN = 128
_H = 64
_NLAYERS = 4
_EPS = 1e-5

_NC = 2
_NS = 16
_NW = _NC * _NS
_BE = 128
_NB = 80
_EPW = _NB * _BE
_EPAD = _NW * _EPW
_NACC = 10112
_RPT = _NACC // _NS
_K = 8
_HBM_SLOTS = 5


def _sc_aggr_body(m_hbm, edges_hbm, zro_hbm, out_hbm,
                  src_v, dst_v, rows_v, acc_sh, gsem, ssem):
    c = lax.axis_index("c")
    s = lax.axis_index("s")
    wid = c * _NS + s
    pltpu.sync_copy(edges_hbm.at[0, wid], src_v)
    pltpu.sync_copy(edges_hbm.at[1, wid], dst_v)
    pltpu.sync_copy(zro_hbm.at[pl.ds(s * _RPT, _RPT)],
                    acc_sh.at[pl.ds(s * _RPT, _RPT)])
    plsc.subcore_barrier()

    def src_of(b):
        return m_hbm

    for b in range(_K):
        pltpu.async_copy(src_of(b).at[src_v.at[b]], rows_v.at[b], gsem.at[b])

    def chunk(i, carry):
        for b in range(_K):
            j = i * _K + b
            jp = j - 1
            bp = (b - 1) % _K
            pltpu.make_async_copy(src_of(b).at[src_v.at[j]], rows_v.at[b],
                                  gsem.at[b]).wait()
            pltpu.async_copy(rows_v.at[b], acc_sh.at[dst_v.at[j]],
                             ssem.at[b], add=True)

            @pl.when(jnp.logical_and(jp >= 0, jp + _K < _NB))
            def _refill():
                pltpu.make_async_copy(rows_v.at[bp], acc_sh.at[dst_v.at[jp]],
                                      ssem.at[bp]).wait()
                pltpu.async_copy(src_of(bp).at[src_v.at[jp + _K]],
                                 rows_v.at[bp], gsem.at[bp])
        return carry

    lax.fori_loop(0, _NB // _K, chunk, 0)
    for b in range(_K):
        j = _NB - _K + b
        pltpu.make_async_copy(rows_v.at[b], acc_sh.at[dst_v.at[j]],
                              ssem.at[b]).wait()
    plsc.subcore_barrier()
    pltpu.sync_copy(acc_sh.at[pl.ds(s * _RPT, _RPT)],
                    out_hbm.at[c, pl.ds(s * _RPT, _RPT)])


@functools.cache
def _sc_aggr():
    return pl.kernel(
        _sc_aggr_body,
        out_type=jax.ShapeDtypeStruct((_NC, _NACC, _H), jnp.float32),
        mesh=plsc.VectorSubcoreMesh(core_axis_name="c", subcore_axis_name="s"),
        scratch_types=[
            pltpu.VMEM((_NB, _BE), jnp.int32),
            pltpu.VMEM((_NB, _BE), jnp.int32),
            pltpu.VMEM((_K, _BE, _H), jnp.float32),
            pltpu.VMEM_SHARED((_NACC, _H), jnp.float32),
            pltpu.SemaphoreType.DMA((_K,)),
            pltpu.SemaphoreType.DMA((_K,)),
        ],
        compiler_params=pltpu.CompilerParams(use_tc_tiling_on_sc=False),
    )


def _dot(a, b):
    return jnp.dot(a, b, preferred_element_type=jnp.float32)


def _tc_in_body(x_ref, wi_ref, bi_ref, w1_ref, b1_ref, w2_ref, b2_ref,
                h_ref, m_ref):
    h = jnp.maximum(_dot(x_ref[...], wi_ref[...]) + bi_ref[...], 0.0)
    h_ref[...] = h
    t = jnp.maximum(_dot(h, w1_ref[...]) + b1_ref[...], 0.0)
    m_ref[...] = _dot(t, w2_ref[...]) + b2_ref[...]


def _gru_bn(h, part, wih_r, wih_z, wih_n, bih_r, bih_z, bih_n,
            whh_r, whh_z, whh_n, bhh_r, bhh_z, bhh_n, gam, bet):
    aggr = part[0, :_N] + part[1, :_N]
    r = jax.nn.sigmoid(_dot(aggr, wih_r) + bih_r + _dot(h, whh_r) + bhh_r)
    z = jax.nn.sigmoid(_dot(aggr, wih_z) + bih_z + _dot(h, whh_z) + bhh_z)
    n = jnp.tanh(_dot(aggr, wih_n) + bih_n + r * (_dot(h, whh_n) + bhh_n))
    hn = (1.0 - z) * n + z * h
    mean = jnp.mean(hn, axis=0, keepdims=True)
    var = jnp.mean((hn - mean) ** 2, axis=0, keepdims=True)
    return (hn - mean) / jnp.sqrt(var + _EPS) * gam + bet + h


def _tc_up_body(h_ref, part_ref, wih_r_ref, wih_z_ref, wih_n_ref,
                bih_r_ref, bih_z_ref, bih_n_ref, whh_r_ref, whh_z_ref,
                whh_n_ref, bhh_r_ref, bhh_z_ref, bhh_n_ref, gam_ref, bet_ref,
                w1_ref, b1_ref, w2_ref, b2_ref, h_out_ref, m_out_ref):
    hb = _gru_bn(h_ref[...], part_ref[...],
                 wih_r_ref[...], wih_z_ref[...], wih_n_ref[...],
                 bih_r_ref[...], bih_z_ref[...], bih_n_ref[...],
                 whh_r_ref[...], whh_z_ref[...], whh_n_ref[...],
                 bhh_r_ref[...], bhh_z_ref[...], bhh_n_ref[...],
                 gam_ref[...], bet_ref[...])
    h_out_ref[...] = hb
    t = jnp.maximum(_dot(hb, w1_ref[...]) + b1_ref[...], 0.0)
    m_out_ref[...] = _dot(t, w2_ref[...]) + b2_ref[...]


def _tc_last_body(h_ref, part_ref, wih_r_ref, wih_z_ref, wih_n_ref,
                  bih_r_ref, bih_z_ref, bih_n_ref, whh_r_ref, whh_z_ref,
                  whh_n_ref, bhh_r_ref, bhh_z_ref, bhh_n_ref, gam_ref,
                  bet_ref, wo_ref, bo_ref, out_ref):
    hb = _gru_bn(h_ref[...], part_ref[...],
                 wih_r_ref[...], wih_z_ref[...], wih_n_ref[...],
                 bih_r_ref[...], bih_z_ref[...], bih_n_ref[...],
                 whh_r_ref[...], whh_z_ref[...], whh_n_ref[...],
                 bhh_r_ref[...], bhh_z_ref[...], bhh_n_ref[...],
                 gam_ref[...], bet_ref[...])
    out_ref[...] = _dot(hb, wo_ref[...]) + bo_ref[...]


def _f32_out(shape):
    return jax.ShapeDtypeStruct(shape, jnp.float32)


@functools.cache
def _tc_in():
    return pl.pallas_call(
        _tc_in_body,
        out_shape=[_f32_out((_N, _H)), _f32_out((_N, _H))],
    )


@functools.cache
def _tc_up():
    return pl.pallas_call(
        _tc_up_body,
        out_shape=[_f32_out((_N, _H)), _f32_out((_N, _H))],
    )


@functools.cache
def _tc_last():
    return pl.pallas_call(
        _tc_last_body,
        out_shape=_f32_out((_N, 1)),
    )


def _split_gru(lp):
    w_ih, w_hh = lp["gru_w_ih"], lp["gru_w_hh"]
    b_ih, b_hh = lp["gru_b_ih"], lp["gru_b_hh"]
    parts = []
    for k in range(3):
        parts.append(w_ih[k * _H:(k + 1) * _H].T)
    for k in range(3):
        parts.append(b_ih[k * _H:(k + 1) * _H].reshape(1, _H))
    for k in range(3):
        parts.append(w_hh[k * _H:(k + 1) * _H].T)
    for k in range(3):
        parts.append(b_hh[k * _H:(k + 1) * _H].reshape(1, _H))
    return (parts[0], parts[1], parts[2], parts[3], parts[4], parts[5],
            parts[6], parts[7], parts[8], parts[9], parts[10], parts[11],
            lp["bn_gamma"].reshape(1, _H), lp["bn_beta"].reshape(1, _H))


def kernel(x, edge_index, params):
    src = edge_index[0]
    dst = edge_index[1]
    pad = _EPAD - _E
    pad_src = jnp.arange(pad, dtype=jnp.int32) % _N
    pad_dst = _N + (jnp.arange(pad, dtype=jnp.int32) % (_NACC - _N))
    edges_p = jnp.concatenate(
        [edge_index, jnp.stack([pad_src, pad_dst])],
        axis=1).reshape(2, _NW, _NB, _BE)
    zeros_acc = jnp.zeros((_NACC, _H), jnp.float32)

    layers = params["layers"]
    lp0 = layers[0]
    h, m = _tc_in()(
        x, params["input_w"], params["input_b"].reshape(1, _H),
        lp0["msg_w1"], lp0["msg_b1"].reshape(1, _H),
        lp0["msg_w2"], lp0["msg_b2"].reshape(1, _H))

    out = None
    for li in range(_NLAYERS):
        lp = layers[li]
        part = _sc_aggr()(m, edges_p, zeros_acc)
        gru = _split_gru(lp)
        if li < _NLAYERS - 1:
            lpn = layers[li + 1]
            h, m = _tc_up()(
                h, part, *gru,
                lpn["msg_w1"], lpn["msg_b1"].reshape(1, _H),
                lpn["msg_w2"], lpn["msg_b2"].reshape(1, _H))
        else:
            out = _tc_last()(
                h, part, *gru,
                params["out_w"], params["out_b"].reshape(1, 1))
    return jnp.squeeze(out, axis=-1)
